# baseline (speedup 1.0000x reference)
import functools

import jax
import jax.numpy as jnp
from jax import lax
from jax.experimental import pallas as pl
from jax.experimental.pallas import tpu as pltpu
from jax.experimental.pallas import tpu_sc as plsc

B = 64
TI = 256
CJ = 2048
EC = 128


def _edge_agg_sc(x, src, dst):
    n, d = x.shape
    e = src.shape[0]
    info = plsc.get_sparse_core_info()
    nc, ns = info.num_cores, info.num_subcores
    nw = nc * ns
    npad = ((n + 8 * ns - 1) // (8 * ns)) * (8 * ns)
    rps = npad // ns

    epw = -(-e // nw)
    epw = ((epw + EC - 1) // EC) * EC
    nch = epw // EC
    padlen = epw * nw - e
    if padlen:
        src = jnp.concatenate([src, jnp.zeros((padlen,), jnp.int32)])
        dst = jnp.concatenate([dst, jnp.full((padlen,), n, jnp.int32)])

    src3 = src.reshape(nw, nch, EC)
    dst3 = dst.reshape(nw, nch, EC)
    zero = jnp.zeros((rps, d), jnp.float32)

    mesh = plsc.VectorSubcoreMesh(core_axis_name="c", subcore_axis_name="s")

    @functools.partial(
        pl.kernel,
        mesh=mesh,
        compiler_params=pltpu.CompilerParams(needs_layout_passes=False),
        out_type=jax.ShapeDtypeStruct((nc * npad, d), jnp.float32),
        scratch_types=[
            pltpu.VMEM((nch, EC), jnp.int32),
            pltpu.VMEM((nch, EC), jnp.int32),
            pltpu.VMEM((EC, d), jnp.float32),
            pltpu.VMEM_SHARED((npad, d), jnp.float32),
            pltpu.SemaphoreType.DMA,
        ],
    )
    def agg(x_hbm, src_hbm, dst_hbm, zero_hbm, out_hbm,
            src_v, dst_v, rows_v, acc_sh, sem):
        cid = lax.axis_index("c")
        sid = lax.axis_index("s")
        wid = cid * ns + sid
        pltpu.sync_copy(zero_hbm, acc_sh.at[pl.ds(sid * rps, rps)])
        pltpu.sync_copy(src_hbm.at[wid], src_v)
        pltpu.sync_copy(dst_hbm.at[wid], dst_v)
        plsc.subcore_barrier()

        def body(t, carry):
            pltpu.async_copy(x_hbm.at[src_v.at[t]], rows_v, sem).wait()
            pltpu.sync_copy(rows_v, acc_sh.at[dst_v.at[t]], add=True)
            return carry

        lax.fori_loop(0, nch, body, 0)
        plsc.subcore_barrier()
        pltpu.sync_copy(acc_sh.at[pl.ds(sid * rps, rps)],
                        out_hbm.at[pl.ds(cid * npad + sid * rps, rps)])

    out = agg(x, src3, dst3, zero)
    return jnp.concatenate([out[0:n], out[npad:npad + n]], axis=0)


def _k2_score(partials, x, w_rel, w_root, b_rel):
    n, d = x.shape

    def body(p_ref, x_ref, wa_ref, wb_ref, b_ref, s_ref):
        aggv = p_ref[0:n, :] + p_ref[n:2 * n, :]
        pre = (jnp.dot(aggv, wa_ref[...], preferred_element_type=jnp.float32)
               + b_ref[0, 0]
               + jnp.dot(x_ref[...], wb_ref[...],
                         preferred_element_type=jnp.float32))
        s_ref[...] = jnp.tanh(pre)

    return pl.pallas_call(
        body,
        out_shape=jax.ShapeDtypeStruct((n, 1), jnp.float32),
    )(partials, x, w_rel, w_root, b_rel.reshape(1, 1))


def _k3_pool(score_col, batch_col, score_row, batch_row, xp):
    np_, d = xp.shape
    nt = np_ // TI
    ncj = np_ // CJ

    def body(sc_ref, bc_ref, sr_ref, br_ref, x_ref, out_ref, cnt_ref):
        i = pl.program_id(0)
        s_i = sc_ref[...]
        b_i = bc_ref[...]
        row_ids = i * TI + lax.broadcasted_iota(jnp.int32, (TI, 1), 0)

        def chunk(c, carry):
            rank, cnt = carry
            s_j = sr_ref[:, pl.ds(c * CJ, CJ)]
            b_j = br_ref[:, pl.ds(c * CJ, CJ)]
            col_ids = c * CJ + lax.broadcasted_iota(jnp.int32, (1, CJ), 1)
            same = b_j == b_i
            beats = (s_j > s_i) | ((s_j == s_i) & (col_ids < row_ids))
            rank = rank + jnp.sum(jnp.where(same & beats, 1.0, 0.0),
                                  axis=1, keepdims=True)
            cnt = cnt + jnp.sum(jnp.where(same, 1.0, 0.0),
                                axis=1, keepdims=True)
            return rank, cnt

        rank, cnt = lax.fori_loop(
            0, ncj, chunk,
            (jnp.zeros((TI, 1), jnp.float32), jnp.zeros((TI, 1), jnp.float32)))

        k = jnp.floor((cnt + 1.0) * 0.5)
        keep = (rank < k) & (b_i < B)
        multv = jnp.where(keep, s_i, 0.0)
        xw = x_ref[...] * multv

        b_i_row = br_ref[:, pl.ds(i * TI, TI)]
        bins = lax.broadcasted_iota(jnp.int32, (B, 1), 0)
        ind = (bins == b_i_row).astype(jnp.float32)
        contrib = jnp.dot(ind, xw, preferred_element_type=jnp.float32)
        cntc = jnp.dot(ind, keep.astype(jnp.float32),
                       preferred_element_type=jnp.float32)

        @pl.when(i == 0)
        def _():
            out_ref[...] = jnp.zeros_like(out_ref)
            cnt_ref[...] = jnp.zeros_like(cnt_ref)

        out_ref[...] += contrib
        cnt_ref[...] += cntc

        @pl.when(i == nt - 1)
        def _():
            out_ref[...] = out_ref[...] / jnp.maximum(cnt_ref[...], 1.0)

    return pl.pallas_call(
        body,
        grid=(nt,),
        in_specs=[
            pl.BlockSpec((TI, 1), lambda i: (i, 0)),
            pl.BlockSpec((TI, 1), lambda i: (i, 0)),
            pl.BlockSpec((1, np_), lambda i: (0, 0)),
            pl.BlockSpec((1, np_), lambda i: (0, 0)),
            pl.BlockSpec((TI, d), lambda i: (i, 0)),
        ],
        out_specs=pl.BlockSpec((B, d), lambda i: (0, 0)),
        out_shape=jax.ShapeDtypeStruct((B, d), jnp.float32),
        scratch_shapes=[pltpu.VMEM((B, 1), jnp.float32)],
    )(score_col, batch_col, score_row, batch_row, xp)


def kernel(x, edge_index, batch, W_rel, b_rel, W_root):
    n, d = x.shape
    np_ = ((n + CJ - 1) // CJ) * CJ

    src = edge_index[0].astype(jnp.int32)
    dst = edge_index[1].astype(jnp.int32)
    partials = _edge_agg_sc(x, src, dst)

    score = _k2_score(partials, x, W_rel, W_root, b_rel.astype(jnp.float32))

    pad = np_ - n
    score_col = jnp.pad(score, ((0, pad), (0, 0)))
    score_row = score_col.reshape(1, np_)
    batch_p = jnp.pad(batch.astype(jnp.int32), (0, pad),
                      constant_values=jnp.int32(1 << 20))
    batch_row = batch_p.reshape(1, np_)
    batch_col = batch_p.reshape(np_, 1)
    xp = jnp.pad(x, ((0, pad), (0, 0)))

    return _k3_pool(score_col, batch_col, score_row, batch_row, xp)


# speedup vs baseline: 1.1760x; 1.1760x over previous
import functools

import jax
import jax.numpy as jnp
from jax import lax
from jax.experimental import pallas as pl
from jax.experimental.pallas import tpu as pltpu
from jax.experimental.pallas import tpu_sc as plsc

B = 64
TI = 256
CJ = 2048
EC = 80


def _edge_agg_sc(x, src, dst):
    n, d = x.shape
    e = src.shape[0]
    info = plsc.get_sparse_core_info()
    nc, ns = info.num_cores, info.num_subcores
    nw = nc * ns
    npad = ((n + 8 * ns - 1) // (8 * ns)) * (8 * ns)
    rps = npad // ns

    epw = -(-e // nw)
    epw = ((epw + EC - 1) // EC) * EC
    nch = epw // EC
    padlen = epw * nw - e
    if padlen:
        src = jnp.concatenate([src, jnp.zeros((padlen,), jnp.int32)])
        dst = jnp.concatenate([dst, jnp.full((padlen,), n, jnp.int32)])

    src3 = src.reshape(nw, nch, EC)
    dst3 = dst.reshape(nw, nch, EC)
    zero = jnp.zeros((rps, d), jnp.float32)

    mesh = plsc.VectorSubcoreMesh(core_axis_name="c", subcore_axis_name="s")

    @functools.partial(
        pl.kernel,
        mesh=mesh,
        compiler_params=pltpu.CompilerParams(needs_layout_passes=False),
        out_type=jax.ShapeDtypeStruct((nc * npad, d), jnp.float32),
        scratch_types=[
            pltpu.VMEM((nch, EC), jnp.int32),
            pltpu.VMEM((nch, EC), jnp.int32),
            pltpu.VMEM((EC, d), jnp.float32),
            pltpu.VMEM_SHARED((npad, d), jnp.float32),
            pltpu.SemaphoreType.DMA,
        ],
    )
    def agg(x_hbm, src_hbm, dst_hbm, zero_hbm, out_hbm,
            src_v, dst_v, rows_v, acc_sh, sem):
        cid = lax.axis_index("c")
        sid = lax.axis_index("s")
        wid = cid * ns + sid
        pltpu.sync_copy(zero_hbm, acc_sh.at[pl.ds(sid * rps, rps)])
        pltpu.sync_copy(src_hbm.at[wid], src_v)
        pltpu.sync_copy(dst_hbm.at[wid], dst_v)
        plsc.subcore_barrier()

        def body(t, carry):
            pltpu.async_copy(x_hbm.at[src_v.at[t]], rows_v, sem).wait()
            pltpu.sync_copy(rows_v, acc_sh.at[dst_v.at[t]], add=True)
            return carry

        lax.fori_loop(0, nch, body, 0)
        plsc.subcore_barrier()
        pltpu.sync_copy(acc_sh.at[pl.ds(sid * rps, rps)],
                        out_hbm.at[pl.ds(cid * npad + sid * rps, rps)])

    out = agg(x, src3, dst3, zero)
    return jnp.concatenate([out[0:n], out[npad:npad + n]], axis=0)


def _k2_score(partials, x, w_rel, w_root, b_rel):
    n, d = x.shape

    def body(p_ref, x_ref, wa_ref, wb_ref, b_ref, s_ref):
        aggv = p_ref[0:n, :] + p_ref[n:2 * n, :]
        pre = (jnp.dot(aggv, wa_ref[...], preferred_element_type=jnp.float32)
               + b_ref[0, 0]
               + jnp.dot(x_ref[...], wb_ref[...],
                         preferred_element_type=jnp.float32))
        s_ref[...] = jnp.tanh(pre)

    return pl.pallas_call(
        body,
        out_shape=jax.ShapeDtypeStruct((n, 1), jnp.float32),
    )(partials, x, w_rel, w_root, b_rel.reshape(1, 1))


def _k3_pool(score_col, batch_col, score_row, batch_row, xp):
    np_, d = xp.shape
    nt = np_ // TI
    ncj = np_ // CJ

    def body(sc_ref, bc_ref, sr_ref, br_ref, x_ref, out_ref, cnt_ref):
        i = pl.program_id(0)
        s_i = sc_ref[...]
        b_i = bc_ref[...]
        row_ids = i * TI + lax.broadcasted_iota(jnp.int32, (TI, 1), 0)

        def chunk(c, carry):
            rank, cnt = carry
            s_j = sr_ref[:, pl.ds(c * CJ, CJ)]
            b_j = br_ref[:, pl.ds(c * CJ, CJ)]
            col_ids = c * CJ + lax.broadcasted_iota(jnp.int32, (1, CJ), 1)
            same = b_j == b_i
            beats = (s_j > s_i) | ((s_j == s_i) & (col_ids < row_ids))
            rank = rank + jnp.sum(jnp.where(same & beats, 1.0, 0.0),
                                  axis=1, keepdims=True)
            cnt = cnt + jnp.sum(jnp.where(same, 1.0, 0.0),
                                axis=1, keepdims=True)
            return rank, cnt

        rank, cnt = lax.fori_loop(
            0, ncj, chunk,
            (jnp.zeros((TI, 1), jnp.float32), jnp.zeros((TI, 1), jnp.float32)))

        k = jnp.floor((cnt + 1.0) * 0.5)
        keep = (rank < k) & (b_i < B)
        multv = jnp.where(keep, s_i, 0.0)
        xw = x_ref[...] * multv

        b_i_row = br_ref[:, pl.ds(i * TI, TI)]
        bins = lax.broadcasted_iota(jnp.int32, (B, 1), 0)
        ind = (bins == b_i_row).astype(jnp.float32)
        contrib = jnp.dot(ind, xw, preferred_element_type=jnp.float32)
        cntc = jnp.dot(ind, keep.astype(jnp.float32),
                       preferred_element_type=jnp.float32)

        @pl.when(i == 0)
        def _():
            out_ref[...] = jnp.zeros_like(out_ref)
            cnt_ref[...] = jnp.zeros_like(cnt_ref)

        out_ref[...] += contrib
        cnt_ref[...] += cntc

        @pl.when(i == nt - 1)
        def _():
            out_ref[...] = out_ref[...] / jnp.maximum(cnt_ref[...], 1.0)

    return pl.pallas_call(
        body,
        grid=(nt,),
        in_specs=[
            pl.BlockSpec((TI, 1), lambda i: (i, 0)),
            pl.BlockSpec((TI, 1), lambda i: (i, 0)),
            pl.BlockSpec((1, np_), lambda i: (0, 0)),
            pl.BlockSpec((1, np_), lambda i: (0, 0)),
            pl.BlockSpec((TI, d), lambda i: (i, 0)),
        ],
        out_specs=pl.BlockSpec((B, d), lambda i: (0, 0)),
        out_shape=jax.ShapeDtypeStruct((B, d), jnp.float32),
        scratch_shapes=[pltpu.VMEM((B, 1), jnp.float32)],
    )(score_col, batch_col, score_row, batch_row, xp)


def kernel(x, edge_index, batch, W_rel, b_rel, W_root):
    n, d = x.shape
    np_ = ((n + CJ - 1) // CJ) * CJ

    src = edge_index[0].astype(jnp.int32)
    dst = edge_index[1].astype(jnp.int32)
    partials = _edge_agg_sc(x, src, dst)

    score = _k2_score(partials, x, W_rel, W_root, b_rel.astype(jnp.float32))

    pad = np_ - n
    score_col = jnp.pad(score, ((0, pad), (0, 0)))
    score_row = score_col.reshape(1, np_)
    batch_p = jnp.pad(batch.astype(jnp.int32), (0, pad),
                      constant_values=jnp.int32(1 << 20))
    batch_row = batch_p.reshape(1, np_)
    batch_col = batch_p.reshape(np_, 1)
    xp = jnp.pad(x, ((0, pad), (0, 0)))

    return _k3_pool(score_col, batch_col, score_row, batch_row, xp)
